# baseline (speedup 1.0000x reference)
import functools

import jax
import jax.numpy as jnp
from jax import lax
from jax.experimental import pallas as pl
from jax.experimental.pallas import tpu as pltpu, tpu_sc as plsc

K = 26
VOCAB = 100000
D = 32
B = 16384

NC = 2
NS = 16
L = 16
NW = NC * NS
ROWS_PER_W = B // NW
BB = 64
NBLK = ROWS_PER_W // BB
IDX_PER_BLK = BB * K
MCH = IDX_PER_BLK // 128


def _emb_kernel(x_hbm, offs_hbm, table_hbm, out_hbm,
                xblk_v, offs_v, idx_v, rows_v, acc_v, sem):
    wid = lax.axis_index("s") * NC + lax.axis_index("c")
    base = wid * ROWS_PER_W

    pltpu.sync_copy(offs_hbm, offs_v)

    def block_body(blk, carry):
        row0 = base + blk * BB

        pltpu.sync_copy(x_hbm.at[pl.ds(row0 * K, IDX_PER_BLK)], xblk_v)

        for m in range(MCH):
            for j in range(128 // L):
                sl = pl.ds(j * L, L)
                idx_v[m, sl] = xblk_v[pl.ds((m * 8 + j) * L, L)] + offs_v[m, sl]

        copies = []
        for m in range(MCH):
            copies.append(
                pltpu.async_copy(
                    table_hbm.at[idx_v.at[m]],
                    rows_v.at[pl.ds(m * 128, 128)],
                    sem,
                )
            )
        for c in copies:
            c.wait()

        def reduce_body(b, carry2):
            p0 = b * K
            acc0 = rows_v[p0, pl.ds(0, L)]
            acc1 = rows_v[p0, pl.ds(L, L)]
            for k in range(1, K):
                acc0 = acc0 + rows_v[p0 + k, pl.ds(0, L)]
                acc1 = acc1 + rows_v[p0 + k, pl.ds(L, L)]
            acc_v[b, pl.ds(0, L)] = acc0
            acc_v[b, pl.ds(L, L)] = acc1
            return carry2

        lax.fori_loop(0, BB, reduce_body, 0)

        pltpu.sync_copy(acc_v, out_hbm.at[pl.ds(row0, BB)])
        return carry

    lax.fori_loop(0, NBLK, block_body, 0)


def kernel(x, tables):
    x_flat = x.reshape(B * K)
    table_flat = tables.reshape(K * VOCAB, D)
    offs = jnp.tile(jnp.arange(K, dtype=jnp.int32) * VOCAB, BB).reshape(MCH, 128)

    mesh = plsc.VectorSubcoreMesh(core_axis_name="c", subcore_axis_name="s")
    f = functools.partial(
        pl.kernel,
        mesh=mesh,
        out_type=jax.ShapeDtypeStruct((B, D), jnp.float32),
        scratch_types=[
            pltpu.VMEM((IDX_PER_BLK,), jnp.int32),
            pltpu.VMEM((MCH, 128), jnp.int32),
            pltpu.VMEM((MCH, 128), jnp.int32),
            pltpu.VMEM((IDX_PER_BLK, D), jnp.float32),
            pltpu.VMEM((BB, D), jnp.float32),
            pltpu.SemaphoreType.DMA,
        ],
        compiler_params=pltpu.CompilerParams(use_tc_tiling_on_sc=False),
    )(_emb_kernel)
    return f(x_flat, offs, table_flat)
